# baseline (speedup 1.0000x reference)
import functools

import jax
import jax.numpy as jnp
from jax import lax
from jax.experimental import pallas as pl
from jax.experimental.pallas import tpu as pltpu
from jax.experimental.pallas import tpu_sc as plsc

N_ROWS = 16384
N_COLS = 512
N_TOTAL = N_ROWS * N_COLS

NC = 2
NS = 16
LANES = 16
NW = NC * NS

PER_W = N_TOTAL // NW
CHUNK = 16384
NCHUNKS = PER_W // CHUNK
VECS_PER_CHUNK = CHUNK // LANES


def _loss_body(x_hbm, t_hbm, out_hbm, xb0, xb1, tb0, tb1, accv, sem0, sem1):
    wid = lax.axis_index("s") * NC + lax.axis_index("c")
    base = wid * PER_W

    xbufs = (xb0, xb1)
    tbufs = (tb0, tb1)
    sems = (sem0, sem1)
    copies = [None, None]

    def start(g):
        slot = g % 2
        off = base + g * CHUNK
        cx = pltpu.async_copy(x_hbm.at[pl.ds(off, CHUNK)], xbufs[slot], sems[slot])
        ct = pltpu.async_copy(t_hbm.at[pl.ds(off, CHUNK)], tbufs[slot], sems[slot])
        copies[slot] = (cx, ct)

    start(0)

    zero = jnp.zeros((LANES,), jnp.float32)
    tot = (zero, zero, zero)

    for g in range(NCHUNKS):
        slot = g % 2
        if g + 1 < NCHUNKS:
            start(g + 1)
        cx, ct = copies[slot]
        cx.wait()
        ct.wait()

        xbuf = xbufs[slot]
        tbuf = tbufs[slot]

        def chunk_body(i, acc, xbuf=xbuf, tbuf=tbuf):
            ap, an, at = acc
            xv = xbuf[pl.ds(i * LANES, LANES)]
            tv = tbuf[pl.ds(i * LANES, LANES)]
            m = jnp.maximum(xv, 0.0)
            z = jnp.exp(-jnp.abs(xv))
            w = z / (z + 2.0)
            w2 = w * w
            poly = 1.0 + w2 * (
                0.3333333333 + w2 * (0.2 + w2 * (0.1428571429 + w2 * 0.1111111111))
            )
            s = m + 2.0 * w * poly
            bp = jnp.minimum(s - xv, 100.0)
            bn = jnp.minimum(s, 100.0)
            ap = ap + tv * bp
            an = an + (bn - tv * bn)
            at = at + tv
            return (ap, an, at)

        ca = lax.fori_loop(0, VECS_PER_CHUNK, chunk_body, (zero, zero, zero))
        tot = (tot[0] + ca[0], tot[1] + ca[1], tot[2] + ca[2])

    accv[pl.ds(0, LANES)] = tot[0]
    accv[pl.ds(LANES, LANES)] = tot[1]
    accv[pl.ds(2 * LANES, LANES)] = tot[2]
    pltpu.sync_copy(accv, out_hbm.at[wid])


@jax.jit
def _sc_loss(x_flat, t_flat):
    mesh = plsc.VectorSubcoreMesh(core_axis_name="c", subcore_axis_name="s")
    fn = pl.kernel(
        _loss_body,
        out_type=jax.ShapeDtypeStruct((NW, 3 * LANES), jnp.float32),
        mesh=mesh,
        scratch_types=[
            pltpu.VMEM((CHUNK,), jnp.float32),
            pltpu.VMEM((CHUNK,), jnp.float32),
            pltpu.VMEM((CHUNK,), jnp.float32),
            pltpu.VMEM((CHUNK,), jnp.float32),
            pltpu.VMEM((3 * LANES,), jnp.float32),
            pltpu.SemaphoreType.DMA,
            pltpu.SemaphoreType.DMA,
        ],
    )
    return fn(x_flat, t_flat)


def kernel(font_output_data, font_target_data):
    x = font_output_data.reshape(-1)
    t = font_target_data.reshape(-1)
    partials = _sc_loss(x, t)
    sums = partials.reshape(NW, 3, LANES).sum(axis=(0, 2))
    pos_sum, neg_sum, t_sum = sums[0], sums[1], sums[2]
    pos_count = jnp.maximum(t_sum, 1.0)
    neg_count = jnp.maximum(jnp.float32(N_TOTAL) - t_sum, 1.0)
    pos_loss = 0.5 * pos_sum / pos_count
    neg_loss = 0.5 * neg_sum / neg_count
    return (pos_loss + neg_loss, pos_loss, neg_loss)


# speedup vs baseline: 2.1414x; 2.1414x over previous
import functools

import jax
import jax.numpy as jnp
from jax import lax
from jax.experimental import pallas as pl
from jax.experimental.pallas import tpu as pltpu
from jax.experimental.pallas import tpu_sc as plsc

N_ROWS = 16384
N_COLS = 512
N_TOTAL = N_ROWS * N_COLS

NC = 2
NS = 16
LANES = 16
NW = NC * NS

PER_W = N_TOTAL // NW
CHUNK = 16384
NCHUNKS = PER_W // CHUNK
VECS_PER_CHUNK = CHUNK // LANES


def _loss_body(x_hbm, t_hbm, out_hbm, xb0, xb1, tb0, tb1, accv, sem0, sem1):
    wid = lax.axis_index("s") * NC + lax.axis_index("c")
    base = wid * PER_W

    xbufs = (xb0, xb1)
    tbufs = (tb0, tb1)
    sems = (sem0, sem1)
    copies = [None, None]

    def start(g):
        slot = g % 2
        off = base + g * CHUNK
        cx = pltpu.async_copy(x_hbm.at[pl.ds(off, CHUNK)], xbufs[slot], sems[slot])
        ct = pltpu.async_copy(t_hbm.at[pl.ds(off, CHUNK)], tbufs[slot], sems[slot])
        copies[slot] = (cx, ct)

    start(0)

    zero = jnp.zeros((LANES,), jnp.float32)
    tot = (zero, zero, zero)

    for g in range(NCHUNKS):
        slot = g % 2
        if g + 1 < NCHUNKS:
            start(g + 1)
        cx, ct = copies[slot]
        cx.wait()
        ct.wait()

        xbuf = xbufs[slot]
        tbuf = tbufs[slot]

        def chunk_body(i, acc, xbuf=xbuf, tbuf=tbuf):
            ap, an, at = acc
            xv = xbuf[pl.ds(i * LANES, LANES)]
            tv = tbuf[pl.ds(i * LANES, LANES)]
            m = jnp.maximum(xv, 0.0)
            z = jnp.exp(-jnp.abs(xv))
            w = z / (z + 2.0)
            w2 = w * w
            poly = 1.0 + w2 * (
                0.3333333333 + w2 * (0.2 + w2 * (0.1428571429 + w2 * 0.1111111111))
            )
            s = m + 2.0 * w * poly
            bp = jnp.minimum(s - xv, 100.0)
            bn = jnp.minimum(s, 100.0)
            ap = ap + tv * bp
            an = an + (bn - tv * bn)
            at = at + tv
            return (ap, an, at)

        ca = lax.fori_loop(0, VECS_PER_CHUNK, chunk_body, (zero, zero, zero))
        tot = (tot[0] + ca[0], tot[1] + ca[1], tot[2] + ca[2])

    accv[pl.ds(0, LANES)] = tot[0]
    accv[pl.ds(LANES, LANES)] = tot[1]
    accv[pl.ds(2 * LANES, LANES)] = tot[2]
    pltpu.sync_copy(accv, out_hbm.at[wid])


@jax.jit
def _sc_loss(x_flat, t_flat):
    mesh = plsc.VectorSubcoreMesh(core_axis_name="c", subcore_axis_name="s")
    fn = pl.kernel(
        _loss_body,
        out_type=jax.ShapeDtypeStruct((NW, 3 * LANES), jnp.float32),
        mesh=mesh,
        scratch_types=[
            pltpu.VMEM((CHUNK,), jnp.float32),
            pltpu.VMEM((CHUNK,), jnp.float32),
            pltpu.VMEM((CHUNK,), jnp.float32),
            pltpu.VMEM((CHUNK,), jnp.float32),
            pltpu.VMEM((3 * LANES,), jnp.float32),
            pltpu.SemaphoreType.DMA,
            pltpu.SemaphoreType.DMA,
        ],
    )
    return fn(x_flat, t_flat)


TC_BLOCK_ROWS = 512


def _tc_body(x_ref, t_ref, out_ref, acc_ref):
    i = pl.program_id(0)

    @pl.when(i == 0)
    def _():
        acc_ref[...] = jnp.zeros_like(acc_ref)

    x = x_ref[...]
    t = t_ref[...]
    m = jnp.maximum(x, 0.0)
    s = m + jnp.log1p(jnp.exp(-jnp.abs(x)))
    bp = jnp.minimum(s - x, 100.0)
    bn = jnp.minimum(s, 100.0)
    pos = (t * bp).reshape(TC_BLOCK_ROWS // 8, 8, N_COLS // 128, 128).sum(axis=(0, 2))
    neg = (bn - t * bn).reshape(TC_BLOCK_ROWS // 8, 8, N_COLS // 128, 128).sum(axis=(0, 2))
    cnt = t.reshape(TC_BLOCK_ROWS // 8, 8, N_COLS // 128, 128).sum(axis=(0, 2))
    acc_ref[0] += pos
    acc_ref[1] += neg
    acc_ref[2] += cnt

    @pl.when(i == pl.num_programs(0) - 1)
    def _():
        out_ref[...] = acc_ref[...]


@jax.jit
def _tc_loss(x, t):
    rows = x.shape[0]
    grid = rows // TC_BLOCK_ROWS
    return pl.pallas_call(
        _tc_body,
        grid=(grid,),
        in_specs=[
            pl.BlockSpec((TC_BLOCK_ROWS, N_COLS), lambda i: (i, 0)),
            pl.BlockSpec((TC_BLOCK_ROWS, N_COLS), lambda i: (i, 0)),
        ],
        out_specs=pl.BlockSpec((3, 8, 128), lambda i: (0, 0, 0)),
        out_shape=jax.ShapeDtypeStruct((3, 8, 128), jnp.float32),
        scratch_shapes=[pltpu.VMEM((3, 8, 128), jnp.float32)],
    )(x, t)


def kernel(font_output_data, font_target_data):
    x = font_output_data
    t = font_target_data
    part = _tc_loss(x, t)
    sums = part.sum(axis=(1, 2))
    pos_sum, neg_sum, t_sum = sums[0], sums[1], sums[2]
    pos_count = jnp.maximum(t_sum, 1.0)
    neg_count = jnp.maximum(jnp.float32(N_TOTAL) - t_sum, 1.0)
    pos_loss = 0.5 * pos_sum / pos_count
    neg_loss = 0.5 * neg_sum / neg_count
    return (pos_loss + neg_loss, pos_loss, neg_loss)
